# baseline (speedup 1.0000x reference)
import jax
import jax.numpy as jnp
from jax.experimental import pallas as pl
from jax.experimental.pallas import tpu as pltpu

B, NT, NS = 16, 4096, 1024
CT, CS = 256, 512
C1, C2 = 256, 256
TB = 2048


def _fp_kernel(xt_ref, xst_ref, ft_ref, fs_ref, w1a_ref, w1b_ref, w2_ref,
               out_ref, g_scr):
    @pl.when(pl.program_id(1) == 0)
    def _():
        g_scr[...] = jnp.dot(fs_ref[0], w1a_ref[...],
                             preferred_element_type=jnp.float32)
    xt = xt_ref[0]
    xst = xst_ref[0]
    diff0 = xt[:, 0:1] - xst[0:1, :]
    d2 = diff0 * diff0
    diff1 = xt[:, 1:2] - xst[1:2, :]
    d2 = d2 + diff1 * diff1
    diff2 = xt[:, 2:3] - xst[2:3, :]
    d2 = d2 + diff2 * diff2

    d2w = d2
    recips = []
    wselu = None
    for k in range(3):
        m = jnp.min(d2w, axis=1, keepdims=True)
        oh = d2w == m
        r = jax.lax.rsqrt(jnp.maximum(m, 1e-20))
        recips.append(r)
        wselu = jnp.where(oh, r, 0.0 if wselu is None else wselu)
        if k < 2:
            d2w = jnp.where(oh, jnp.float32(jnp.inf), d2w)
    r0, r1, r2 = recips
    norm = r0 + r1 + r2
    rn = 1.0 / norm
    ws = (r0 + r1 + r2) * rn + 1e-6
    c = rn / ws

    interp = c * jnp.dot(wselu, g_scr[...],
                         preferred_element_type=jnp.float32)
    h = interp + jnp.dot(ft_ref[0], w1b_ref[...],
                         preferred_element_type=jnp.float32)
    h = jnp.maximum(h, 0.0)
    out = jnp.dot(h, w2_ref[...], preferred_element_type=jnp.float32)
    out_ref[0] = jnp.maximum(out, 0.0)


@jax.jit
def kernel(xyz_target, xyz_source, feats_target, feats_source, W1, W2):
    W1a = W1[:CS]
    W1b = W1[CS:]
    xst = jnp.swapaxes(xyz_source, 1, 2)

    out = pl.pallas_call(
        _fp_kernel,
        grid=(B, NT // TB),
        in_specs=[
            pl.BlockSpec((1, TB, 3), lambda b, t: (b, t, 0)),
            pl.BlockSpec((1, 3, NS), lambda b, t: (b, 0, 0)),
            pl.BlockSpec((1, TB, CT), lambda b, t: (b, t, 0)),
            pl.BlockSpec((1, NS, CS), lambda b, t: (b, 0, 0)),
            pl.BlockSpec((CS, C1), lambda b, t: (0, 0)),
            pl.BlockSpec((CT, C1), lambda b, t: (0, 0)),
            pl.BlockSpec((C1, C2), lambda b, t: (0, 0)),
        ],
        out_specs=pl.BlockSpec((1, TB, C2), lambda b, t: (b, t, 0)),
        out_shape=jax.ShapeDtypeStruct((B, NT, C2), jnp.float32),
        scratch_shapes=[pltpu.VMEM((NS, C1), jnp.float32)],
    )(xyz_target, xst, feats_target, feats_source, W1a, W1b, W2)
    return out
